# Initial kernel scaffold; baseline (speedup 1.0000x reference)
import jax
import jax.numpy as jnp
from jax.experimental import pallas as pl


def kernel(x, M1, M2):
    raise NotImplementedError("write your pallas kernel here")


# speedup vs baseline: 18.8613x; 18.8613x over previous
import functools

import jax
import jax.numpy as jnp
from jax import lax
from jax.experimental import pallas as pl
from jax.experimental.pallas import tpu as pltpu
from jax.experimental.pallas import tpu_sc as plsc

_N = 4096
_D = 64
_KTOP = 167772
_TOTAL = _N * _N

_NC = 2
_NS = 16
_NW = _NC * _NS
_CHUNK = _TOTAL // _NW
_WIN = 65536
_NWIN = _CHUNK // _WIN


def _v_body(m1_ref, m2_ref, v_ref):
    s = lax.dot_general(m1_ref[...], m2_ref[...], (((1,), (1,)), ((), ())),
                        preferred_element_type=jnp.float32)
    v_ref[...] = jnp.maximum(s, 0.0)


def _compute_v(M1, M2):
    blk = 256
    return pl.pallas_call(
        _v_body,
        grid=(_N // blk,),
        in_specs=[pl.BlockSpec((blk, _D), lambda i: (i, 0)),
                  pl.BlockSpec((_N, _D), lambda i: (0, 0))],
        out_specs=pl.BlockSpec((blk, _N), lambda i: (i, 0)),
        out_shape=jax.ShapeDtypeStruct((_N, _N), jnp.float32),
    )(M1, M2)


def _make_hist(nbins, mode):
    hist_words = 16 * nbins
    mesh = plsc.VectorSubcoreMesh(core_axis_name="c", subcore_axis_name="s")
    with_sel = mode != 1

    def body(*refs):
        if with_sel:
            v_hbm, r_hbm, out_hbm, win_v, hist_v, sel_v = refs
        else:
            v_hbm, out_hbm, win_v, hist_v = refs
        wid = lax.axis_index("s") * _NC + lax.axis_index("c")
        base = wid * _CHUNK
        lane_base = lax.iota(jnp.int32, 16) * nbins
        ones = jnp.ones((16,), jnp.int32)

        def zbody(i, _):
            hist_v[pl.ds(i * 16, 16)] = jnp.zeros((16,), jnp.int32)
            return 0
        lax.fori_loop(0, hist_words // 16, zbody, 0)

        if with_sel:
            pltpu.sync_copy(r_hbm.at[0, pl.ds(0, 16)], sel_v)
            sel = sel_v[...]

        def win_body(w, _):
            pltpu.sync_copy(v_hbm.at[pl.ds(base + w * _WIN, _WIN)], win_v)

            def ebody(i, _):
                v = win_v[pl.ds(i * 16, 16)]
                bits = lax.bitcast_convert_type(v, jnp.int32)
                if mode == 1:
                    bucket = lax.shift_right_logical(bits, 20)
                    plsc.addupdate_scatter(hist_v, [lane_base + bucket], ones)
                elif mode == 2:
                    bucket = jnp.bitwise_and(
                        lax.shift_right_logical(bits, 9), 0x7FF)
                    mk = lax.shift_right_logical(bits, 20) == sel
                    plsc.addupdate_scatter(hist_v, [lane_base + bucket], ones,
                                           mask=mk)
                else:
                    bucket = jnp.bitwise_and(bits, 0x1FF)
                    mk = lax.shift_right_logical(bits, 9) == sel
                    plsc.addupdate_scatter(hist_v, [lane_base + bucket], ones,
                                           mask=mk)
                return 0
            lax.fori_loop(0, _WIN // 16, ebody, 0)
            return 0
        lax.fori_loop(0, _NWIN, win_body, 0)

        pltpu.sync_copy(hist_v, out_hbm.at[wid])

    scratch = [pltpu.VMEM((_WIN,), jnp.float32),
               pltpu.VMEM((hist_words,), jnp.int32)]
    if with_sel:
        scratch.append(pltpu.VMEM((16,), jnp.int32))
    return pl.kernel(
        body,
        out_type=jax.ShapeDtypeStruct((_NW, hist_words), jnp.int32),
        mesh=mesh,
        scratch_types=scratch,
        compiler_params=pltpu.CompilerParams(needs_layout_passes=False),
    )


def _search(h, nbits, k):
    nbins = h.shape[1]
    j = lax.broadcasted_iota(jnp.int32, (1, nbins), 1)
    p = jnp.int32(0)
    for i in range(nbits):
        c = p + jnp.int32(1 << (nbits - 1 - i))
        ic = jnp.sum(jnp.where(j >= c, h, 0))
        p = jnp.where(ic >= k, c, p)
    ca = jnp.sum(jnp.where(j > p, h, 0))
    return p, k - ca


def _r1_body(h_ref, out_ref):
    h = jnp.sum(h_ref[...], axis=0, keepdims=True)
    b, kp = _search(h, 11, jnp.int32(_KTOP))
    row = lax.broadcasted_iota(jnp.int32, (8, 128), 0)
    out_ref[...] = jnp.where(row == 0, b, kp)


def _r2_body(h_ref, r_ref, out_ref):
    b1 = r_ref[0, 0]
    k1 = r_ref[1, 0]
    h = jnp.sum(h_ref[...], axis=0, keepdims=True)
    b2, k2 = _search(h, 11, k1)
    c2 = jnp.bitwise_or(lax.shift_left(b1, 11), b2)
    row = lax.broadcasted_iota(jnp.int32, (8, 128), 0)
    out_ref[...] = jnp.where(row == 0, c2, k2)


def _r3_body(h_ref, r_ref, out_ref):
    c2 = r_ref[0, 0]
    k2 = r_ref[1, 0]
    h = jnp.sum(h_ref[...], axis=0, keepdims=True)
    b3, _ = _search(h, 9, k2)
    tbits = jnp.bitwise_or(lax.shift_left(c2, 9), b3)
    t = lax.bitcast_convert_type(jnp.full((8, 128), tbits, jnp.int32),
                                 jnp.float32)
    out_ref[...] = jax.nn.sigmoid(t)


def _reduce1(h1):
    return pl.pallas_call(
        _r1_body,
        out_shape=jax.ShapeDtypeStruct((8, 128), jnp.int32),
    )(h1)


def _reduce2(h2, r1):
    return pl.pallas_call(
        _r2_body,
        out_shape=jax.ShapeDtypeStruct((8, 128), jnp.int32),
    )(h2, r1)


def _reduce3(h3, r2):
    return pl.pallas_call(
        _r3_body,
        out_shape=jax.ShapeDtypeStruct((8, 128), jnp.float32),
    )(h3, r2)


def _mask_body(v_ref, t_ref, out_ref):
    thr = t_ref[0, 0]
    a = jax.nn.sigmoid(v_ref[...])
    out_ref[...] = jnp.where(a > thr, a, 0.0)


def _apply_mask(V, thr):
    blk = 256
    return pl.pallas_call(
        _mask_body,
        grid=(_N // blk,),
        in_specs=[pl.BlockSpec((blk, _N), lambda i: (i, 0)),
                  pl.BlockSpec((8, 128), lambda i: (0, 0))],
        out_specs=pl.BlockSpec((blk, _N), lambda i: (i, 0)),
        out_shape=jax.ShapeDtypeStruct((_N, _N), jnp.float32),
    )(V, thr)


_hist1 = _make_hist(2048, 1)
_hist2 = _make_hist(2048, 2)
_hist3 = _make_hist(512, 3)


def kernel(x, M1, M2):
    del x
    V = _compute_v(M1, M2)
    vf = V.reshape(_TOTAL)
    h1 = _hist1(vf).reshape(_NW * 16, 2048)
    r1 = _reduce1(h1)
    h2 = _hist2(vf, r1).reshape(_NW * 16, 2048)
    r2 = _reduce2(h2, r1)
    h3 = _hist3(vf, r2).reshape(_NW * 16, 512)
    thr = _reduce3(h3, r2)
    return _apply_mask(V, thr)


# speedup vs baseline: 24.5694x; 1.3026x over previous
import functools

import jax
import jax.numpy as jnp
from jax import lax
from jax.experimental import pallas as pl
from jax.experimental.pallas import tpu as pltpu
from jax.experimental.pallas import tpu_sc as plsc

_N = 4096
_D = 64
_KTOP = 167772
_TOTAL = _N * _N

_NC = 2
_NS = 16
_NW = _NC * _NS
_CHUNK = _TOTAL // _NW
_WIN = 32768
_NWIN = _CHUNK // _WIN
_UNROLL = 8


def _v_body(m1_ref, m2_ref, v_ref):
    s = lax.dot_general(m1_ref[...], m2_ref[...], (((1,), (1,)), ((), ())),
                        preferred_element_type=jnp.float32)
    v_ref[...] = jnp.maximum(s, 0.0)


def _compute_v(M1, M2):
    blk = 256
    return pl.pallas_call(
        _v_body,
        grid=(_N // blk,),
        in_specs=[pl.BlockSpec((blk, _D), lambda i: (i, 0)),
                  pl.BlockSpec((_N, _D), lambda i: (0, 0))],
        out_specs=pl.BlockSpec((blk, _N), lambda i: (i, 0)),
        out_shape=jax.ShapeDtypeStruct((_N, _N), jnp.float32),
    )(M1, M2)


def _make_hist(nbins, mode):
    hist_words = nbins * 16
    mesh = plsc.VectorSubcoreMesh(core_axis_name="c", subcore_axis_name="s")
    with_sel = mode != 1

    def body(*refs):
        if with_sel:
            v_hbm, r_hbm, out_hbm, win0, win1, hist_v, sel_v, sem0, sem1 = refs
        else:
            v_hbm, out_hbm, win0, win1, hist_v, sem0, sem1 = refs
        wid = lax.axis_index("s") * _NC + lax.axis_index("c")
        base = wid * _CHUNK
        lane = lax.iota(jnp.int32, 16)
        ones = jnp.ones((16,), jnp.int32)

        def zbody(i, _):
            for u in range(_UNROLL):
                hist_v[pl.ds((i * _UNROLL + u) * 16, 16)] = (
                    jnp.zeros((16,), jnp.int32))
            return 0
        lax.fori_loop(0, hist_words // (16 * _UNROLL), zbody, 0)

        if with_sel:
            pltpu.sync_copy(r_hbm.at[0, pl.ds(0, 16)], sel_v)
            sel = sel_v[...]
        else:
            sel = None

        def update(win, i):
            b0 = i * (16 * _UNROLL)
            for u in range(_UNROLL):
                v = win[pl.ds(b0 + u * 16, 16)]
                bits = lax.bitcast_convert_type(v, jnp.int32)
                if mode == 1:
                    bucket = lax.shift_right_logical(bits, 20)
                    mk = None
                elif mode == 2:
                    bucket = jnp.bitwise_and(
                        lax.shift_right_logical(bits, 9), 0x7FF)
                    mk = lax.shift_right_logical(bits, 20) == sel
                else:
                    bucket = jnp.bitwise_and(bits, 0x1FF)
                    mk = lax.shift_right_logical(bits, 9) == sel
                idx = jnp.bitwise_or(lax.shift_left(bucket, 4), lane)
                plsc.addupdate_scatter(hist_v, [idx], ones, mask=mk)

        def process(win):
            def ibody(i, _):
                update(win, i)
                return 0
            lax.fori_loop(0, _WIN // (16 * _UNROLL), ibody, 0)

        def dma(w, buf, sem):
            return pltpu.make_async_copy(
                v_hbm.at[pl.ds(base + w * _WIN, _WIN)], buf, sem)

        dma(0, win0, sem0).start()

        def pair_body(p, _):
            w0 = 2 * p
            dma(w0 + 1, win1, sem1).start()
            dma(w0, win0, sem0).wait()
            process(win0)

            @pl.when(p < _NWIN // 2 - 1)
            def _():
                dma(w0 + 2, win0, sem0).start()

            dma(w0 + 1, win1, sem1).wait()
            process(win1)
            return 0
        lax.fori_loop(0, _NWIN // 2, pair_body, 0)

        pltpu.sync_copy(hist_v, out_hbm.at[wid])

    scratch = [pltpu.VMEM((_WIN,), jnp.float32),
               pltpu.VMEM((_WIN,), jnp.float32),
               pltpu.VMEM((hist_words,), jnp.int32)]
    if with_sel:
        scratch.append(pltpu.VMEM((16,), jnp.int32))
    scratch += [pltpu.SemaphoreType.DMA, pltpu.SemaphoreType.DMA]
    return pl.kernel(
        body,
        out_type=jax.ShapeDtypeStruct((_NW, hist_words), jnp.int32),
        mesh=mesh,
        scratch_types=scratch,
        compiler_params=pltpu.CompilerParams(needs_layout_passes=False),
    )


def _search(h, jb, nbits, k):
    p = jnp.int32(0)
    for i in range(nbits):
        c = p + jnp.int32(1 << (nbits - 1 - i))
        ic = jnp.sum(jnp.where(jb >= c, h, 0))
        p = jnp.where(ic >= k, c, p)
    ca = jnp.sum(jnp.where(jb > p, h, 0))
    return p, k - ca


def _hist_cols(h_ref):
    h = jnp.sum(h_ref[...], axis=0, keepdims=True)
    j = lax.broadcasted_iota(jnp.int32, h.shape, 1)
    return h, lax.shift_right_logical(j, 4)


def _r1_body(h_ref, out_ref):
    h, jb = _hist_cols(h_ref)
    b, kp = _search(h, jb, 11, jnp.int32(_KTOP))
    row = lax.broadcasted_iota(jnp.int32, (8, 128), 0)
    out_ref[...] = jnp.where(row == 0, b, kp)


def _r2_body(h_ref, r_ref, out_ref):
    b1 = r_ref[0, 0]
    k1 = r_ref[1, 0]
    h, jb = _hist_cols(h_ref)
    b2, k2 = _search(h, jb, 11, k1)
    c2 = jnp.bitwise_or(lax.shift_left(b1, 11), b2)
    row = lax.broadcasted_iota(jnp.int32, (8, 128), 0)
    out_ref[...] = jnp.where(row == 0, c2, k2)


def _r3_body(h_ref, r_ref, out_ref):
    c2 = r_ref[0, 0]
    k2 = r_ref[1, 0]
    h, jb = _hist_cols(h_ref)
    b3, _ = _search(h, jb, 9, k2)
    tbits = jnp.bitwise_or(lax.shift_left(c2, 9), b3)
    t = lax.bitcast_convert_type(jnp.full((8, 128), tbits, jnp.int32),
                                 jnp.float32)
    out_ref[...] = jax.nn.sigmoid(t)


def _reduce1(h1):
    return pl.pallas_call(
        _r1_body,
        out_shape=jax.ShapeDtypeStruct((8, 128), jnp.int32),
    )(h1)


def _reduce2(h2, r1):
    return pl.pallas_call(
        _r2_body,
        out_shape=jax.ShapeDtypeStruct((8, 128), jnp.int32),
    )(h2, r1)


def _reduce3(h3, r2):
    return pl.pallas_call(
        _r3_body,
        out_shape=jax.ShapeDtypeStruct((8, 128), jnp.float32),
    )(h3, r2)


def _mask_body(v_ref, t_ref, out_ref):
    thr = t_ref[0, 0]
    a = jax.nn.sigmoid(v_ref[...])
    out_ref[...] = jnp.where(a > thr, a, 0.0)


def _apply_mask(V, thr):
    blk = 256
    return pl.pallas_call(
        _mask_body,
        grid=(_N // blk,),
        in_specs=[pl.BlockSpec((blk, _N), lambda i: (i, 0)),
                  pl.BlockSpec((8, 128), lambda i: (0, 0))],
        out_specs=pl.BlockSpec((blk, _N), lambda i: (i, 0)),
        out_shape=jax.ShapeDtypeStruct((_N, _N), jnp.float32),
    )(V, thr)


_hist1 = _make_hist(2048, 1)
_hist2 = _make_hist(2048, 2)
_hist3 = _make_hist(512, 3)


def kernel(x, M1, M2):
    del x
    V = _compute_v(M1, M2)
    vf = V.reshape(_TOTAL)
    h1 = _hist1(vf)
    r1 = _reduce1(h1)
    h2 = _hist2(vf, r1)
    r2 = _reduce2(h2, r1)
    h3 = _hist3(vf, r2)
    thr = _reduce3(h3, r2)
    return _apply_mask(V, thr)


# speedup vs baseline: 76.9653x; 3.1326x over previous
import functools

import jax
import jax.numpy as jnp
from jax import lax
from jax.experimental import pallas as pl
from jax.experimental.pallas import tpu as pltpu
from jax.experimental.pallas import tpu_sc as plsc

_N = 4096
_D = 64
_KTOP = 167772
_TOTAL = _N * _N

_NC = 2
_NS = 16
_NW = _NC * _NS
_CHUNK = _TOTAL // _NW
_WIN = 32768
_NWIN = _CHUNK // _WIN
_UNROLL = 8


def _v_body(m1_ref, m2_ref, v_ref):
    s = lax.dot_general(m1_ref[...], m2_ref[...], (((1,), (1,)), ((), ())),
                        preferred_element_type=jnp.float32)
    v_ref[...] = jnp.maximum(s, 0.0)


def _compute_v(M1, M2):
    blk = 256
    return pl.pallas_call(
        _v_body,
        grid=(_N // blk,),
        in_specs=[pl.BlockSpec((blk, _D), lambda i: (i, 0)),
                  pl.BlockSpec((_N, _D), lambda i: (0, 0))],
        out_specs=pl.BlockSpec((blk, _N), lambda i: (i, 0)),
        out_shape=jax.ShapeDtypeStruct((_N, _N), jnp.float32),
    )(M1, M2)


def _make_hist(nbins, mode):
    hist_words = nbins * 16
    mesh = plsc.VectorSubcoreMesh(core_axis_name="c", subcore_axis_name="s")
    with_sel = mode != 1

    def body(*refs):
        if with_sel:
            v_hbm, r_hbm, out_hbm, win0, win1, hist_v, sel_v, sem0, sem1 = refs
        else:
            v_hbm, out_hbm, win0, win1, hist_v, sem0, sem1 = refs
        wid = lax.axis_index("s") * _NC + lax.axis_index("c")
        base = wid * _CHUNK
        lane = lax.iota(jnp.int32, 16)
        ones = jnp.ones((16,), jnp.int32)

        def zbody(i, _):
            for u in range(_UNROLL):
                hist_v[pl.ds((i * _UNROLL + u) * 16, 16)] = (
                    jnp.zeros((16,), jnp.int32))
            return 0
        lax.fori_loop(0, hist_words // (16 * _UNROLL), zbody, 0)

        if with_sel:
            pltpu.sync_copy(r_hbm.at[0, pl.ds(0, 16)], sel_v)
            sel = sel_v[...]
        else:
            sel = None

        def process(win):
            @plsc.parallel_loop(0, _WIN // 16, unroll=_UNROLL)
            def _(i):
                v = win[pl.ds(i * 16, 16)]
                bits = lax.bitcast_convert_type(v, jnp.int32)
                if mode == 1:
                    bucket = lax.shift_right_logical(bits, 20)
                    mk = None
                elif mode == 2:
                    bucket = jnp.bitwise_and(
                        lax.shift_right_logical(bits, 9), 0x7FF)
                    mk = lax.shift_right_logical(bits, 20) == sel
                else:
                    bucket = jnp.bitwise_and(bits, 0x1FF)
                    mk = lax.shift_right_logical(bits, 9) == sel
                idx = jnp.bitwise_or(lax.shift_left(bucket, 4), lane)
                plsc.addupdate_scatter(hist_v, [idx], ones, mask=mk)

        def dma(w, buf, sem):
            return pltpu.make_async_copy(
                v_hbm.at[pl.ds(base + w * _WIN, _WIN)], buf, sem)

        dma(0, win0, sem0).start()

        def pair_body(p, _):
            w0 = 2 * p
            dma(w0 + 1, win1, sem1).start()
            dma(w0, win0, sem0).wait()
            process(win0)

            @pl.when(p < _NWIN // 2 - 1)
            def _():
                dma(w0 + 2, win0, sem0).start()

            dma(w0 + 1, win1, sem1).wait()
            process(win1)
            return 0
        lax.fori_loop(0, _NWIN // 2, pair_body, 0)

        pltpu.sync_copy(hist_v, out_hbm.at[wid])

    scratch = [pltpu.VMEM((_WIN,), jnp.float32),
               pltpu.VMEM((_WIN,), jnp.float32),
               pltpu.VMEM((hist_words,), jnp.int32)]
    if with_sel:
        scratch.append(pltpu.VMEM((16,), jnp.int32))
    scratch += [pltpu.SemaphoreType.DMA, pltpu.SemaphoreType.DMA]
    return pl.kernel(
        body,
        out_type=jax.ShapeDtypeStruct((_NW, hist_words), jnp.int32),
        mesh=mesh,
        scratch_types=scratch,
        compiler_params=pltpu.CompilerParams(needs_layout_passes=False),
    )


def _search(h, jb, nbits, k):
    p = jnp.int32(0)
    for i in range(nbits):
        c = p + jnp.int32(1 << (nbits - 1 - i))
        ic = jnp.sum(jnp.where(jb >= c, h, 0))
        p = jnp.where(ic >= k, c, p)
    ca = jnp.sum(jnp.where(jb > p, h, 0))
    return p, k - ca


def _hist_cols(h_ref):
    h = jnp.sum(h_ref[...], axis=0, keepdims=True)
    j = lax.broadcasted_iota(jnp.int32, h.shape, 1)
    return h, lax.shift_right_logical(j, 4)


def _r1_body(h_ref, out_ref):
    h, jb = _hist_cols(h_ref)
    b, kp = _search(h, jb, 11, jnp.int32(_KTOP))
    row = lax.broadcasted_iota(jnp.int32, (8, 128), 0)
    out_ref[...] = jnp.where(row == 0, b, kp)


def _r2_body(h_ref, r_ref, out_ref):
    b1 = r_ref[0, 0]
    k1 = r_ref[1, 0]
    h, jb = _hist_cols(h_ref)
    b2, k2 = _search(h, jb, 11, k1)
    c2 = jnp.bitwise_or(lax.shift_left(b1, 11), b2)
    row = lax.broadcasted_iota(jnp.int32, (8, 128), 0)
    out_ref[...] = jnp.where(row == 0, c2, k2)


def _r3_body(h_ref, r_ref, out_ref):
    c2 = r_ref[0, 0]
    k2 = r_ref[1, 0]
    h, jb = _hist_cols(h_ref)
    b3, _ = _search(h, jb, 9, k2)
    tbits = jnp.bitwise_or(lax.shift_left(c2, 9), b3)
    t = lax.bitcast_convert_type(jnp.full((8, 128), tbits, jnp.int32),
                                 jnp.float32)
    out_ref[...] = jax.nn.sigmoid(t)


def _reduce1(h1):
    return pl.pallas_call(
        _r1_body,
        out_shape=jax.ShapeDtypeStruct((8, 128), jnp.int32),
    )(h1)


def _reduce2(h2, r1):
    return pl.pallas_call(
        _r2_body,
        out_shape=jax.ShapeDtypeStruct((8, 128), jnp.int32),
    )(h2, r1)


def _reduce3(h3, r2):
    return pl.pallas_call(
        _r3_body,
        out_shape=jax.ShapeDtypeStruct((8, 128), jnp.float32),
    )(h3, r2)


def _mask_body(v_ref, t_ref, out_ref):
    thr = t_ref[0, 0]
    a = jax.nn.sigmoid(v_ref[...])
    out_ref[...] = jnp.where(a > thr, a, 0.0)


def _apply_mask(V, thr):
    blk = 256
    return pl.pallas_call(
        _mask_body,
        grid=(_N // blk,),
        in_specs=[pl.BlockSpec((blk, _N), lambda i: (i, 0)),
                  pl.BlockSpec((8, 128), lambda i: (0, 0))],
        out_specs=pl.BlockSpec((blk, _N), lambda i: (i, 0)),
        out_shape=jax.ShapeDtypeStruct((_N, _N), jnp.float32),
    )(V, thr)


_hist1 = _make_hist(2048, 1)
_hist2 = _make_hist(2048, 2)
_hist3 = _make_hist(512, 3)


def kernel(x, M1, M2):
    del x
    V = _compute_v(M1, M2)
    vf = V.reshape(_TOTAL)
    h1 = _hist1(vf)
    r1 = _reduce1(h1)
    h2 = _hist2(vf, r1)
    r2 = _reduce2(h2, r1)
    h3 = _hist3(vf, r2)
    thr = _reduce3(h3, r2)
    return _apply_mask(V, thr)
